# Initial kernel scaffold; baseline (speedup 1.0000x reference)
import jax
import jax.numpy as jnp
from jax.experimental import pallas as pl


def kernel(features, feature_values, emb_table, bias_table, W1, b1, W2, b2, Wp, bp):
    raise NotImplementedError("write your pallas kernel here")


# speedup vs baseline: 1.2328x; 1.2328x over previous
import functools

import jax
import jax.numpy as jnp
from jax import lax
from jax.experimental import pallas as pl
from jax.experimental.pallas import tpu as pltpu
from jax.experimental.pallas import tpu_sc as plsc

B = 16384
F = 26
V = 1000000
D = 16

NC = 2
NS = 16
NW = NC * NS
NB = B * F
PER_W = NB // NW
CH = 1024
N_CH = PER_W // CH


def _sc_gather_body(emb_hbm, bias_hbm, idx_hbm, rows_out, bias_out,
                    idx_v, rows_v, bias_v, sem_r, sem_b):
    wid = lax.axis_index("s") * NC + lax.axis_index("c")
    base = wid * PER_W

    def body(i, carry):
        off = base + i * CH
        pltpu.sync_copy(idx_hbm.at[pl.ds(off, CH)], idx_v)
        cp_r = pltpu.async_copy(emb_hbm.at[idx_v], rows_v, sem_r)
        cp_b = pltpu.async_copy(bias_hbm.at[idx_v], bias_v, sem_b)
        cp_r.wait()
        cp_b.wait()
        pltpu.sync_copy(rows_v, rows_out.at[pl.ds(off, CH)])
        pltpu.sync_copy(bias_v, bias_out.at[pl.ds(off, CH)])
        return carry

    lax.fori_loop(0, N_CH, body, 0)


_sc_gather = functools.partial(
    pl.kernel,
    out_type=[
        jax.ShapeDtypeStruct((NB, D), jnp.float32),
        jax.ShapeDtypeStruct((NB,), jnp.float32),
    ],
    mesh=plsc.VectorSubcoreMesh(core_axis_name="c", subcore_axis_name="s"),
    scratch_types=[
        pltpu.VMEM((CH,), jnp.int32),
        pltpu.VMEM((CH, D), jnp.float32),
        pltpu.VMEM((CH,), jnp.float32),
        pltpu.SemaphoreType.DMA,
        pltpu.SemaphoreType.DMA,
    ],
    compiler_params=pltpu.CompilerParams(use_tc_tiling_on_sc=False),
)(_sc_gather_body)


BM = 256


def _tc_body(scal_ref, rows_ref, fv_ref, fb_ref, e_ref, p_ref,
             w1t_ref, b1_ref, w2t_ref, b2_ref, wph_ref, out_ref):
    fv = fv_ref[...]
    fve = jnp.dot(fv, e_ref[...],
                  preferred_element_type=jnp.float32)
    fe = rows_ref[...] * fve
    s1 = jnp.dot(fe, p_ref[...], preferred_element_type=jnp.float32)
    s2 = jnp.dot(fe * fe, p_ref[...], preferred_element_type=jnp.float32)
    second = 0.5 * jnp.sum(s1 * s1 - s2, axis=1, keepdims=True)
    first = jnp.sum(fb_ref[...] * fv, axis=1, keepdims=True)
    h = jnp.dot(fe, w1t_ref[...], preferred_element_type=jnp.float32)
    h = jnp.maximum(h + b1_ref[...], 0.0)
    h = jnp.dot(h, w2t_ref[...], preferred_element_type=jnp.float32)
    h = jnp.maximum(h + b2_ref[...], 0.0)
    o = jnp.dot(h, wph_ref[...], preferred_element_type=jnp.float32)
    out_ref[...] = (o + first * scal_ref[0] + second * scal_ref[1]
                    + scal_ref[2])


def _tc_fused(rows, fv, fb, e_mat, p_mat, w1t, b1r, w2t, b2r, wph, scal):
    grid = (B // BM,)
    full = lambda shape: pl.BlockSpec(shape, lambda i: (0, 0))
    return pl.pallas_call(
        _tc_body,
        grid=grid,
        in_specs=[
            pl.BlockSpec(memory_space=pltpu.SMEM),
            pl.BlockSpec((BM, F * D), lambda i: (i, 0)),
            pl.BlockSpec((BM, F), lambda i: (i, 0)),
            pl.BlockSpec((BM, F), lambda i: (i, 0)),
            full((F, F * D)),
            full((F * D, D)),
            full((F * D, 256)),
            full((1, 256)),
            full((256, 128)),
            full((1, 128)),
            full((128, 1)),
        ],
        out_specs=pl.BlockSpec((BM, 1), lambda i: (i, 0)),
        out_shape=jax.ShapeDtypeStruct((B, 1), jnp.float32),
    )(scal, rows, fv, fb, e_mat, p_mat, w1t, b1r, w2t, b2r, wph)


def kernel(features, feature_values, emb_table, bias_table,
           W1, b1, W2, b2, Wp, bp):
    feat = features.reshape(-1).astype(jnp.int32)
    bias_flat = bias_table.reshape(-1)

    rows, fb_flat = _sc_gather(emb_table, bias_flat, feat)

    cols = jnp.arange(F * D, dtype=jnp.int32)
    e_mat = (cols[None, :] // D == jnp.arange(F, dtype=jnp.int32)[:, None]
             ).astype(jnp.float32)
    p_mat = (cols[:, None] % D == jnp.arange(D, dtype=jnp.int32)[None, :]
             ).astype(jnp.float32)
    scal = jnp.concatenate([Wp[0, :2], bp]).astype(jnp.float32)

    out = _tc_fused(rows.reshape(B, F * D), feature_values,
                    fb_flat.reshape(B, F), e_mat, p_mat,
                    W1.T, b1.reshape(1, -1), W2.T, b2.reshape(1, -1),
                    Wp[0, 2:].reshape(-1, 1), scal)
    return out.reshape(-1)
